# baseline (speedup 1.0000x reference)
import functools

import jax
import jax.numpy as jnp
from jax import lax
from jax.experimental import pallas as pl
from jax.experimental.pallas import tpu as pltpu
from jax.experimental.pallas import tpu_sc as plsc

N = 10000
NP = 10240
E = 320000
G = 64
NC, NS = 2, 16
NW = NC * NS
EPW = 10240
ECH = EPW // 128
EP = NW * EPW
DUMMY = N
SORTN = 16384
SLICE = NP // NS
F32, I32 = jnp.float32, jnp.int32
HIGH = lax.Precision.HIGHEST
SDS = jax.ShapeDtypeStruct


def _mesh():
    return plsc.VectorSubcoreMesh(
        core_axis_name="c", subcore_axis_name="s", num_cores=NC, num_subcores=NS)


def _make_sc_edge_prep():
    @functools.partial(
        pl.kernel,
        out_type=(SDS((NW, ECH, 128), I32), SDS((NW, ECH, 128), I32),
                  SDS((NC, NP), F32)),
        mesh=_mesh(),
        scratch_types=[
            pltpu.VMEM((ECH, 128), I32),
            pltpu.VMEM((ECH, 128), I32),
            pltpu.VMEM((128,), F32),
            pltpu.VMEM((128,), F32),
            pltpu.VMEM((128,), F32),
            pltpu.VMEM((SLICE,), F32),
            pltpu.VMEM_SHARED((NP,), F32),
        ],
    )
    def sc_edge_prep(sel_hbm, src_hbm, dst_hbm, zeros_hbm,
                     srco_hbm, dsto_hbm, deg_hbm,
                     src_v, dst_v, kbuf, ssbuf, sdbuf, zbuf, deg_s):
        cid = lax.axis_index("c")
        sid = lax.axis_index("s")
        wid = sid * NC + cid
        pltpu.sync_copy(src_hbm.at[wid], src_v)
        pltpu.sync_copy(dst_hbm.at[wid], dst_v)
        pltpu.sync_copy(zeros_hbm.at[pl.ds(sid * SLICE, SLICE)], zbuf)
        pltpu.sync_copy(zbuf, deg_s.at[pl.ds(sid * SLICE, SLICE)])
        plsc.subcore_barrier()

        def chunk(j, carry):
            pltpu.sync_copy(sel_hbm.at[src_v.at[j]], ssbuf)
            pltpu.sync_copy(sel_hbm.at[dst_v.at[j]], sdbuf)
            for b in range(8):
                sl = pl.ds(b * 16, 16)
                s = src_v[j, sl]
                d = dst_v[j, sl]
                keep = ssbuf[sl] * sdbuf[sl]
                km = keep > 0.0
                src_v[j, sl] = jnp.where(km, s, DUMMY)
                dst_v[j, sl] = jnp.where(km, d, DUMMY)
                kbuf[sl] = keep
            pltpu.sync_copy(kbuf, deg_s.at[dst_v.at[j]], add=True)
            return carry

        lax.fori_loop(0, ECH, chunk, 0)
        plsc.subcore_barrier()
        pltpu.sync_copy(src_v, srco_hbm.at[wid])
        pltpu.sync_copy(dst_v, dsto_hbm.at[wid])
        pltpu.sync_copy(deg_s.at[pl.ds(sid * SLICE, SLICE)], zbuf)
        pltpu.sync_copy(zbuf, deg_hbm.at[cid, pl.ds(sid * SLICE, SLICE)])

    return sc_edge_prep


def _make_sc_rowagg():
    @functools.partial(
        pl.kernel,
        out_type=SDS((NC, NP, 128), F32),
        mesh=_mesh(),
        scratch_types=[
            pltpu.VMEM((ECH, 128), I32),
            pltpu.VMEM((ECH, 128), I32),
            pltpu.VMEM((128, 128), F32),
            pltpu.VMEM_SHARED((NP, 128), F32),
            pltpu.SemaphoreType.DMA,
        ],
    )
    def sc_rowagg(y_hbm, src_hbm, dst_hbm, zeros2d_hbm, acc_hbm,
                  src_v, dst_v, rows0, acc_s, sem):
        cid = lax.axis_index("c")
        sid = lax.axis_index("s")
        wid = sid * NC + cid
        pltpu.sync_copy(src_hbm.at[wid], src_v)
        pltpu.sync_copy(dst_hbm.at[wid], dst_v)
        pltpu.sync_copy(zeros2d_hbm, rows0)
        for q in range(SLICE // 128):
            pltpu.sync_copy(rows0, acc_s.at[pl.ds(sid * SLICE + q * 128, 128)])
        plsc.subcore_barrier()

        def chunk(j, carry):
            pltpu.async_copy(y_hbm.at[src_v.at[j]], rows0, sem).wait()
            pltpu.sync_copy(rows0, acc_s.at[dst_v.at[j]], add=True)
            return carry

        lax.fori_loop(0, ECH, chunk, 0)
        plsc.subcore_barrier()
        for q in range(SLICE // 128):
            r0 = sid * SLICE + q * 128
            pltpu.sync_copy(acc_s.at[pl.ds(r0, 128)], rows0)
            pltpu.sync_copy(rows0, acc_hbm.at[cid, pl.ds(r0, 128)])

    return sc_rowagg


def _make_sc_scatter():
    @functools.partial(
        pl.kernel,
        out_type=(SDS((NP,), F32), SDS((NP,), I32)),
        mesh=_mesh(),
        scratch_types=[
            pltpu.VMEM((4, 128), I32),
            pltpu.VMEM((4, 128), F32),
            pltpu.VMEM((4, 128), I32),
        ],
    )
    def sc_scatter(ids_hbm, sel_hbm_in, pos_hbm_in, sel_hbm, pos_hbm,
                   id_v, sv, pv):
        cid = lax.axis_index("c")
        sid = lax.axis_index("s")
        wid = sid * NC + cid
        pltpu.sync_copy(ids_hbm.at[wid], id_v)
        pltpu.sync_copy(sel_hbm_in.at[wid], sv)
        pltpu.sync_copy(pos_hbm_in.at[wid], pv)
        for q in range(4):
            pltpu.sync_copy(sv.at[q], sel_hbm.at[id_v.at[q]])
            pltpu.sync_copy(pv.at[q], pos_hbm.at[id_v.at[q]])

    return sc_scatter


_BR = 1024


def _tc_xw_y(x, W, deg2):
    def body(x_ref, w_ref, deg_ref, y_ref, dis_ref):
        i = pl.program_id(0)
        deg = deg_ref[0] + deg_ref[1] + 1.0
        dis = 1.0 / jnp.sqrt(deg)
        xw = jnp.dot(x_ref[...], w_ref[...], preferred_element_type=F32)
        rowid = i * _BR + lax.broadcasted_iota(I32, (_BR, 1), 0)
        y_ref[...] = jnp.where(rowid < N, xw * dis, 0.0)
        dis_ref[...] = dis

    return pl.pallas_call(
        body,
        grid=(NP // _BR,),
        in_specs=[
            pl.BlockSpec((_BR, 128), lambda i: (i, 0)),
            pl.BlockSpec((128, 128), lambda i: (0, 0)),
            pl.BlockSpec((2, _BR, 1), lambda i: (0, i, 0)),
        ],
        out_specs=[
            pl.BlockSpec((_BR, 128), lambda i: (i, 0)),
            pl.BlockSpec((_BR, 1), lambda i: (i, 0)),
        ],
        out_shape=[SDS((NP, 128), F32), SDS((NP, 1), F32)],
    )(x, W, deg2)


def _tc_x2(acc, y, dis, bg, Wt):
    def body(acc_ref, y_ref, dis_ref, bg_ref, wt_ref, xp_ref, u_ref):
        i = pl.program_id(0)
        s = acc_ref[0] + acc_ref[1] + y_ref[...]
        xp = jnp.maximum(dis_ref[...] * s + bg_ref[...], 0.0)
        rowid = i * _BR + lax.broadcasted_iota(I32, (_BR, 1), 0)
        xp = jnp.where(rowid < N, xp, 0.0)
        xp_ref[...] = xp
        u_ref[...] = jnp.dot(xp, wt_ref[...], preferred_element_type=F32)

    return pl.pallas_call(
        body,
        grid=(NP // _BR,),
        in_specs=[
            pl.BlockSpec((2, _BR, 128), lambda i: (0, i, 0)),
            pl.BlockSpec((_BR, 128), lambda i: (i, 0)),
            pl.BlockSpec((_BR, 1), lambda i: (i, 0)),
            pl.BlockSpec((1, 128), lambda i: (0, 0)),
            pl.BlockSpec((128, 1), lambda i: (0, 0)),
        ],
        out_specs=[
            pl.BlockSpec((_BR, 128), lambda i: (i, 0)),
            pl.BlockSpec((_BR, 1), lambda i: (i, 0)),
        ],
        out_shape=[SDS((NP, 128), F32), SDS((NP, 1), F32)],
    )(acc, y, dis, bg, Wt)


def _tc_score(agg, Wr, u, br):
    def body(agg_ref, wr_ref, u_ref, br_ref, sc_ref):
        a = agg_ref[0] + agg_ref[1]
        d = jnp.dot(a, wr_ref[...], preferred_element_type=F32)
        sc_ref[...] = jnp.tanh(d + br_ref[0, 0] + u_ref[...])

    return pl.pallas_call(
        body,
        grid=(NP // _BR,),
        in_specs=[
            pl.BlockSpec((2, _BR, 128), lambda i: (0, i, 0)),
            pl.BlockSpec((128, 1), lambda i: (0, 0)),
            pl.BlockSpec((_BR, 1), lambda i: (i, 0)),
            pl.BlockSpec((1, 1), lambda i: (0, 0)),
        ],
        out_specs=[pl.BlockSpec((_BR, 1), lambda i: (i, 0))],
        out_shape=[SDS((NP, 1), F32)],
    )(agg, Wr, u, br)[0]


def _tc_sort(score_g, valid_g, pos_g, batch_g):
    def body(score_ref, val_ref, pos_ref, bat_ref,
             sels_ref, poss_ref, ids_ref):
        score = score_ref[...]
        validb = val_ref[...] > 0.0
        bkey = jnp.where(validb, bat_ref[...], G)
        skey = jnp.where(validb, -score, jnp.inf) + 0.0
        bits = lax.bitcast_convert_type(skey, I32)
        o = jnp.where(bits >= 0, bits ^ jnp.int32(-2147483648), ~bits)
        hi = (bkey << 24) | (jnp.right_shift(o, 8) & jnp.int32(0xFFFFFF))
        lo = ((o & jnp.int32(0xFF)) << 14) | pos_ref[...]
        rows = lax.broadcasted_iota(I32, (128, 128), 0)
        cols = lax.broadcasted_iota(I32, (128, 128), 1)
        pg = rows * 128 + cols
        idv = jnp.minimum(pg, NP - 1)

        def xorperm(x, st):
            if st >= 128:
                m = st // 128
                up = jnp.concatenate([x[m:, :], x[:m, :]], axis=0)
                dn = jnp.concatenate([x[128 - m:, :], x[:128 - m, :]], axis=0)
                bitc = (rows & m) == 0
            else:
                m = st
                up = jnp.concatenate([x[:, m:], x[:, :m]], axis=1)
                dn = jnp.concatenate([x[:, 128 - m:], x[:, :128 - m]], axis=1)
                bitc = (cols & m) == 0
            return jnp.where(bitc, up, dn)

        size = 2
        while size <= SORTN:
            st = size // 2
            while st >= 1:
                hip = xorperm(hi, st)
                lop = xorperm(lo, st)
                idp = xorperm(idv, st)
                eq = hi == hip
                gt = (hi > hip) | (eq & (lo > lop))
                lt = (hi < hip) | (eq & (lo < lop))
                low = (pg & st) == 0
                asc = (pg & size) == 0
                eqm = asc == low
                take = (eqm & gt) | (~eqm & lt)
                hi = jnp.where(take, hip, hi)
                lo = jnp.where(take, lop, lo)
                idv = jnp.where(take, idp, idv)
                st //= 2
            size *= 2

        gsort = jnp.right_shift(hi, 24)
        sel_acc = jnp.zeros((128, 128), F32)
        pos_acc = jnp.full((128, 128), N, I32)
        sstart = jnp.int32(0)
        nstart = jnp.int32(0)
        for gi in range(G):
            c = jnp.sum(jnp.where(bkey == gi, jnp.int32(1), jnp.int32(0)))
            k = (c + 1) // 2
            m = gsort == gi
            rank = pg - sstart
            selm = m & (rank < k)
            sel_acc = jnp.where(selm, 1.0, sel_acc)
            pos_acc = jnp.where(selm, nstart + rank, pos_acc)
            sstart = sstart + c
            nstart = nstart + k
        sels_ref[...] = sel_acc
        poss_ref[...] = pos_acc
        ids_ref[...] = idv

    return pl.pallas_call(
        body,
        out_shape=[SDS((128, 128), F32),
                   SDS((128, 128), I32), SDS((128, 128), I32)],
    )(score_g, valid_g, pos_g, batch_g)


_PBR = 512


def _tc_pool(xp, score_col, sel_col, batch_col):
    nblk = NP // _PBR

    def body(xp_ref, sc_ref, sel_ref, b_ref, xn_ref, hv_ref, hm, hsm):
        i = pl.program_id(0)

        @pl.when(i == 0)
        def _():
            hm[...] = jnp.full((G, 128), -jnp.inf, F32)
            hsm[...] = jnp.zeros((G, 128), F32)

        xn = xp_ref[...] * sc_ref[...] * sel_ref[...]
        xn_ref[...] = xn
        selb = sel_ref[...] > 0.0
        onehot = jnp.where(
            (b_ref[...] == lax.broadcasted_iota(I32, (_PBR, G), 1)) & selb,
            1.0, 0.0)
        hsm[...] += lax.dot_general(onehot, xn, (((0,), (0,)), ((), ())),
                                    precision=HIGH, preferred_element_type=F32)
        for gi in range(G):
            rm = (b_ref[...] == gi) & selb
            vals = jnp.where(rm, xn, -jnp.inf)
            mx = jnp.max(vals, axis=0, keepdims=True)
            hm[gi:gi + 1, :] = jnp.maximum(hm[gi:gi + 1, :], mx)

        @pl.when(i == nblk - 1)
        def _():
            hv_ref[:, :128] = hm[...]
            hv_ref[:, 128:] = hsm[...]

    return pl.pallas_call(
        body,
        grid=(nblk,),
        in_specs=[
            pl.BlockSpec((_PBR, 128), lambda i: (i, 0)),
            pl.BlockSpec((_PBR, 1), lambda i: (i, 0)),
            pl.BlockSpec((_PBR, 1), lambda i: (i, 0)),
            pl.BlockSpec((_PBR, 1), lambda i: (i, 0)),
        ],
        out_specs=[
            pl.BlockSpec((_PBR, 128), lambda i: (i, 0)),
            pl.BlockSpec((G, 256), lambda i: (0, 0)),
        ],
        out_shape=[SDS((NP, 128), F32), SDS((G, 256), F32)],
        scratch_shapes=[pltpu.VMEM((G, 128), F32), pltpu.VMEM((G, 128), F32)],
    )(xp, score_col, sel_col, batch_col)


def _tc_mlp(h0, h1, h2, hls, W0a, W0b, bm0, Wm1, bm1, Wm2, bm2):
    def body(h0_ref, h1_ref, h2_ref, hls_ref, w0a_ref, w0b_ref, b0_ref,
             w1_ref, b1_ref, w2_ref, b2_ref, out_ref):
        h = h0_ref[...] + h1_ref[...] + h2_ref[...]
        z = jnp.dot(h, w0a_ref[...], preferred_element_type=F32)
        z = z + jnp.dot(hls_ref[...], w0b_ref[...],
                        preferred_element_type=F32)
        z = jnp.maximum(z + b0_ref[...], 0.0)
        z = jnp.maximum(jnp.dot(z, w1_ref[...],
                                preferred_element_type=F32) + b1_ref[...], 0.0)
        out_ref[...] = jnp.dot(z, w2_ref[...],
                               preferred_element_type=F32) + b2_ref[...]

    return pl.pallas_call(
        body,
        out_shape=SDS((G, 1), F32),
    )(h0, h1, h2, hls, W0a, W0b, bm0, Wm1, bm1, Wm2, bm2)


def kernel(x, edge_index, batch, hls_attr,
           Wg0, bg0, Wr0, br0, Wt0,
           Wg1, bg1, Wr1, br1, Wt1,
           Wg2, bg2, Wr2, br2, Wt2,
           Wm0, bm0, Wm1, bm1, Wm2, bm2):
    Wg = [Wg0, Wg1, Wg2]
    bg = [bg0, bg1, bg2]
    Wr = [Wr0, Wr1, Wr2]
    br = [br0, br1, br2]
    Wt = [Wt0, Wt1, Wt2]

    sc_edge_prep = _make_sc_edge_prep()
    sc_rowagg = _make_sc_rowagg()
    sc_scatter = _make_sc_scatter()

    batch_i = jnp.asarray(batch).astype(I32)
    src0 = jnp.concatenate(
        [edge_index[0].astype(I32), jnp.full((EP - E,), DUMMY, I32)]
    ).reshape(NW, ECH, 128)
    dst0 = jnp.concatenate(
        [edge_index[1].astype(I32), jnp.full((EP - E,), DUMMY, I32)]
    ).reshape(NW, ECH, 128)
    zeros_np = jnp.zeros((NP,), F32)
    zeros2d = jnp.zeros((128, 128), F32)
    sel01 = jnp.concatenate([jnp.ones((N,), F32), jnp.zeros((NP - N,), F32)])
    posn = jnp.concatenate(
        [jnp.arange(N, dtype=I32), jnp.full((NP - N,), N, I32)])
    batch_g = jnp.concatenate(
        [batch_i, jnp.full((SORTN - N,), G, I32)]).reshape(128, 128)
    batch_col = jnp.concatenate(
        [batch_i, jnp.full((NP - N,), G, I32)]).reshape(NP, 1)

    xcur = jnp.concatenate([x, jnp.zeros((NP - N, 128), F32)], axis=0)
    srcr, dstr = src0, dst0
    hs = []
    for l in range(3):
        srcr, dstr, deg = sc_edge_prep(sel01, srcr, dstr, zeros_np)
        y, dis = _tc_xw_y(xcur, Wg[l], deg.reshape(NC, NP, 1))
        acc = sc_rowagg(y, srcr, dstr, zeros2d)
        xp, u = _tc_x2(acc, y, dis, bg[l].reshape(1, 128), Wt[l])
        agg = sc_rowagg(xp, srcr, dstr, zeros2d)
        score_col = _tc_score(agg, Wr[l], u, br[l].reshape(1, 1))
        score_g = jnp.concatenate(
            [score_col.reshape(NP), jnp.zeros((SORTN - NP,), F32)]
        ).reshape(128, 128)
        valid_g = jnp.concatenate(
            [sel01, jnp.zeros((SORTN - NP,), F32)]).reshape(128, 128)
        pos_g = jnp.concatenate(
            [posn, jnp.full((SORTN - NP,), N, I32)]).reshape(128, 128)
        sel_s, pos_s, ids = _tc_sort(score_g, valid_g, pos_g, batch_g)
        sel01, posn = sc_scatter(
            ids.reshape(NW, 4, 128), sel_s.reshape(NW, 4, 128),
            pos_s.reshape(NW, 4, 128))
        xcur, hv = _tc_pool(xp, score_col, sel01.reshape(NP, 1), batch_col)
        hs.append(hv)

    return _tc_mlp(hs[0], hs[1], hs[2], hls_attr,
                   Wm0[:256], Wm0[256:], bm0.reshape(1, 64),
                   Wm1, bm1.reshape(1, 64), Wm2, bm2.reshape(1, 1))


# speedup vs baseline: 1.0024x; 1.0024x over previous
import functools

import jax
import jax.numpy as jnp
from jax import lax
from jax.experimental import pallas as pl
from jax.experimental.pallas import tpu as pltpu
from jax.experimental.pallas import tpu_sc as plsc

N = 10000
NP = 10240
E = 320000
G = 64
NC, NS = 2, 16
NW = NC * NS
EPW = 10240
ECH = EPW // 128
EP = NW * EPW
DUMMY = N
SORTN = 16384
SLICE = NP // NS
F32, I32 = jnp.float32, jnp.int32
HIGH = lax.Precision.HIGHEST
SDS = jax.ShapeDtypeStruct


def _mesh():
    return plsc.VectorSubcoreMesh(
        core_axis_name="c", subcore_axis_name="s", num_cores=NC, num_subcores=NS)


def _make_sc_edge_prep():
    @functools.partial(
        pl.kernel,
        out_type=(SDS((NW, ECH, 128), I32), SDS((NW, ECH, 128), I32),
                  SDS((NC, NP), F32)),
        mesh=_mesh(),
        scratch_types=[
            pltpu.VMEM((ECH, 128), I32),
            pltpu.VMEM((ECH, 128), I32),
            pltpu.VMEM((EPW,), F32),
            pltpu.VMEM((EPW,), F32),
            pltpu.VMEM((EPW,), F32),
            pltpu.VMEM((SLICE,), F32),
            pltpu.VMEM_SHARED((NP,), F32),
            pltpu.SemaphoreType.DMA,
            pltpu.SemaphoreType.DMA,
        ],
    )
    def sc_edge_prep(sel_hbm, src_hbm, dst_hbm, zeros_hbm,
                     srco_hbm, dsto_hbm, deg_hbm,
                     src_v, dst_v, kbuf, ssbuf, sdbuf, zbuf, deg_s,
                     sem_a, sem_b):
        cid = lax.axis_index("c")
        sid = lax.axis_index("s")
        wid = sid * NC + cid
        pltpu.sync_copy(src_hbm.at[wid], src_v)
        pltpu.sync_copy(dst_hbm.at[wid], dst_v)
        pltpu.sync_copy(zeros_hbm.at[pl.ds(sid * SLICE, SLICE)], zbuf)
        pltpu.sync_copy(zbuf, deg_s.at[pl.ds(sid * SLICE, SLICE)])
        plsc.subcore_barrier()

        def fire(j, carry):
            pltpu.async_copy(sel_hbm.at[src_v.at[j]],
                             ssbuf.at[pl.ds(j * 128, 128)], sem_a)
            pltpu.async_copy(sel_hbm.at[dst_v.at[j]],
                             sdbuf.at[pl.ds(j * 128, 128)], sem_b)
            return carry

        lax.fori_loop(0, ECH, fire, 0)
        pltpu.make_async_copy(zeros_hbm, ssbuf, sem_a).wait()
        pltpu.make_async_copy(zeros_hbm, sdbuf, sem_b).wait()

        def chunk(j, carry):
            for b in range(8):
                sl = pl.ds(b * 16, 16)
                fl = pl.ds(j * 128 + b * 16, 16)
                s = src_v[j, sl]
                d = dst_v[j, sl]
                keep = ssbuf[fl] * sdbuf[fl]
                km = keep > 0.0
                src_v[j, sl] = jnp.where(km, s, DUMMY)
                dst_v[j, sl] = jnp.where(km, d, DUMMY)
                kbuf[fl] = keep
            return carry

        lax.fori_loop(0, ECH, chunk, 0)

        def fire2(j, carry):
            pltpu.async_copy(kbuf.at[pl.ds(j * 128, 128)],
                             deg_s.at[dst_v.at[j]], sem_a, add=True)
            return carry

        lax.fori_loop(0, ECH, fire2, 0)
        pltpu.make_async_copy(zeros_hbm, kbuf, sem_a).wait()
        plsc.subcore_barrier()
        pltpu.sync_copy(src_v, srco_hbm.at[wid])
        pltpu.sync_copy(dst_v, dsto_hbm.at[wid])
        pltpu.sync_copy(deg_s.at[pl.ds(sid * SLICE, SLICE)], zbuf)
        pltpu.sync_copy(zbuf, deg_hbm.at[cid, pl.ds(sid * SLICE, SLICE)])

    return sc_edge_prep


def _make_sc_rowagg():
    @functools.partial(
        pl.kernel,
        out_type=SDS((NC, NP, 128), F32),
        mesh=_mesh(),
        scratch_types=[
            pltpu.VMEM((ECH, 128), I32),
            pltpu.VMEM((1, 128), I32),
            pltpu.VMEM((1, 128), I32),
            pltpu.VMEM((128, 128), F32),
            pltpu.VMEM((128, 128), F32),
            pltpu.VMEM_SHARED((NP, 128), F32),
            pltpu.SemaphoreType.DMA,
            pltpu.SemaphoreType.DMA,
        ],
    )
    def sc_rowagg(y_hbm, src_hbm, dst_hbm, zeros2d_hbm, acc_hbm,
                  src_v, dst_c0, dst_c1, rows0, rows1, acc_s, sem_g0, sem_g1):
        cid = lax.axis_index("c")
        sid = lax.axis_index("s")
        wid = sid * NC + cid
        pltpu.sync_copy(src_hbm.at[wid], src_v)
        pltpu.sync_copy(zeros2d_hbm, rows0)
        for q in range(SLICE // 128):
            pltpu.sync_copy(rows0, acc_s.at[pl.ds(sid * SLICE + q * 128, 128)])
        plsc.subcore_barrier()
        pltpu.async_copy(y_hbm.at[src_v.at[0]], rows0, sem_g0)
        pltpu.sync_copy(dst_hbm.at[wid, pl.ds(0, 1)], dst_c0)

        def pair(g, carry):
            j0 = 2 * g
            j1 = 2 * g + 1
            pltpu.async_copy(y_hbm.at[src_v.at[j1]], rows1, sem_g1)
            pltpu.sync_copy(dst_hbm.at[wid, pl.ds(j1, 1)], dst_c1)
            pltpu.make_async_copy(y_hbm.at[src_v.at[j0]], rows0, sem_g0).wait()
            pltpu.sync_copy(rows0, acc_s.at[dst_c0.at[0]], add=True)

            @pl.when(j0 + 2 < ECH)
            def _():
                pltpu.async_copy(y_hbm.at[src_v.at[j0 + 2]], rows0, sem_g0)
                pltpu.sync_copy(dst_hbm.at[wid, pl.ds(j0 + 2, 1)], dst_c0)

            pltpu.make_async_copy(y_hbm.at[src_v.at[j1]], rows1, sem_g1).wait()
            pltpu.sync_copy(rows1, acc_s.at[dst_c1.at[0]], add=True)
            return carry

        lax.fori_loop(0, ECH // 2, pair, 0)
        plsc.subcore_barrier()
        for q in range(SLICE // 128):
            r0 = sid * SLICE + q * 128
            pltpu.sync_copy(acc_s.at[pl.ds(r0, 128)], rows0)
            pltpu.sync_copy(rows0, acc_hbm.at[cid, pl.ds(r0, 128)])

    return sc_rowagg


def _make_sc_scatter():
    @functools.partial(
        pl.kernel,
        out_type=(SDS((NP,), F32), SDS((NP,), I32)),
        mesh=_mesh(),
        scratch_types=[
            pltpu.VMEM((4, 128), I32),
            pltpu.VMEM((4, 128), F32),
            pltpu.VMEM((4, 128), I32),
            pltpu.SemaphoreType.DMA,
            pltpu.SemaphoreType.DMA,
        ],
    )
    def sc_scatter(ids_hbm, sel_hbm_in, pos_hbm_in, sel_hbm, pos_hbm,
                   id_v, sv, pv, sem_a, sem_b):
        cid = lax.axis_index("c")
        sid = lax.axis_index("s")
        wid = sid * NC + cid
        pltpu.sync_copy(ids_hbm.at[wid], id_v)
        pltpu.sync_copy(sel_hbm_in.at[wid], sv)
        pltpu.sync_copy(pos_hbm_in.at[wid], pv)
        for q in range(4):
            pltpu.async_copy(sv.at[q], sel_hbm.at[id_v.at[q]], sem_a)
            pltpu.async_copy(pv.at[q], pos_hbm.at[id_v.at[q]], sem_b)
        for q in range(4):
            pltpu.make_async_copy(sv.at[q], sel_hbm.at[id_v.at[q]], sem_a).wait()
            pltpu.make_async_copy(pv.at[q], pos_hbm.at[id_v.at[q]], sem_b).wait()

    return sc_scatter


_BR = 1024


def _tc_xw_y(x, W, deg2):
    def body(x_ref, w_ref, deg_ref, y_ref, dis_ref):
        i = pl.program_id(0)
        deg = deg_ref[0] + deg_ref[1] + 1.0
        dis = 1.0 / jnp.sqrt(deg)
        xw = jnp.dot(x_ref[...], w_ref[...], preferred_element_type=F32)
        rowid = i * _BR + lax.broadcasted_iota(I32, (_BR, 1), 0)
        y_ref[...] = jnp.where(rowid < N, xw * dis, 0.0)
        dis_ref[...] = dis

    return pl.pallas_call(
        body,
        grid=(NP // _BR,),
        in_specs=[
            pl.BlockSpec((_BR, 128), lambda i: (i, 0)),
            pl.BlockSpec((128, 128), lambda i: (0, 0)),
            pl.BlockSpec((2, _BR, 1), lambda i: (0, i, 0)),
        ],
        out_specs=[
            pl.BlockSpec((_BR, 128), lambda i: (i, 0)),
            pl.BlockSpec((_BR, 1), lambda i: (i, 0)),
        ],
        out_shape=[SDS((NP, 128), F32), SDS((NP, 1), F32)],
    )(x, W, deg2)


def _tc_x2(acc, y, dis, bg, Wt):
    def body(acc_ref, y_ref, dis_ref, bg_ref, wt_ref, xp_ref, u_ref):
        i = pl.program_id(0)
        s = acc_ref[0] + acc_ref[1] + y_ref[...]
        xp = jnp.maximum(dis_ref[...] * s + bg_ref[...], 0.0)
        rowid = i * _BR + lax.broadcasted_iota(I32, (_BR, 1), 0)
        xp = jnp.where(rowid < N, xp, 0.0)
        xp_ref[...] = xp
        u_ref[...] = jnp.dot(xp, wt_ref[...], preferred_element_type=F32)

    return pl.pallas_call(
        body,
        grid=(NP // _BR,),
        in_specs=[
            pl.BlockSpec((2, _BR, 128), lambda i: (0, i, 0)),
            pl.BlockSpec((_BR, 128), lambda i: (i, 0)),
            pl.BlockSpec((_BR, 1), lambda i: (i, 0)),
            pl.BlockSpec((1, 128), lambda i: (0, 0)),
            pl.BlockSpec((128, 1), lambda i: (0, 0)),
        ],
        out_specs=[
            pl.BlockSpec((_BR, 128), lambda i: (i, 0)),
            pl.BlockSpec((_BR, 1), lambda i: (i, 0)),
        ],
        out_shape=[SDS((NP, 128), F32), SDS((NP, 1), F32)],
    )(acc, y, dis, bg, Wt)


def _tc_score(agg, Wr, u, br):
    def body(agg_ref, wr_ref, u_ref, br_ref, sc_ref):
        a = agg_ref[0] + agg_ref[1]
        d = jnp.dot(a, wr_ref[...], preferred_element_type=F32)
        sc_ref[...] = jnp.tanh(d + br_ref[0, 0] + u_ref[...])

    return pl.pallas_call(
        body,
        grid=(NP // _BR,),
        in_specs=[
            pl.BlockSpec((2, _BR, 128), lambda i: (0, i, 0)),
            pl.BlockSpec((128, 1), lambda i: (0, 0)),
            pl.BlockSpec((_BR, 1), lambda i: (i, 0)),
            pl.BlockSpec((1, 1), lambda i: (0, 0)),
        ],
        out_specs=[pl.BlockSpec((_BR, 1), lambda i: (i, 0))],
        out_shape=[SDS((NP, 1), F32)],
    )(agg, Wr, u, br)[0]


def _tc_sort(score_g, valid_g, pos_g, batch_g):
    def body(score_ref, val_ref, pos_ref, bat_ref,
             sels_ref, poss_ref, ids_ref):
        score = score_ref[...]
        validb = val_ref[...] > 0.0
        bkey = jnp.where(validb, bat_ref[...], G)
        skey = jnp.where(validb, -score, jnp.inf) + 0.0
        bits = lax.bitcast_convert_type(skey, I32)
        o = jnp.where(bits >= 0, bits ^ jnp.int32(-2147483648), ~bits)
        hi = (bkey << 24) | (jnp.right_shift(o, 8) & jnp.int32(0xFFFFFF))
        lo = ((o & jnp.int32(0xFF)) << 14) | pos_ref[...]
        rows = lax.broadcasted_iota(I32, (128, 128), 0)
        cols = lax.broadcasted_iota(I32, (128, 128), 1)
        pg = rows * 128 + cols
        idv = jnp.minimum(pg, NP - 1)

        def xorperm(x, st):
            if st >= 128:
                m = st // 128
                up = jnp.concatenate([x[m:, :], x[:m, :]], axis=0)
                dn = jnp.concatenate([x[128 - m:, :], x[:128 - m, :]], axis=0)
                bitc = (rows & m) == 0
            else:
                m = st
                up = jnp.concatenate([x[:, m:], x[:, :m]], axis=1)
                dn = jnp.concatenate([x[:, 128 - m:], x[:, :128 - m]], axis=1)
                bitc = (cols & m) == 0
            return jnp.where(bitc, up, dn)

        size = 2
        while size <= SORTN:
            st = size // 2
            while st >= 1:
                hip = xorperm(hi, st)
                lop = xorperm(lo, st)
                idp = xorperm(idv, st)
                eq = hi == hip
                gt = (hi > hip) | (eq & (lo > lop))
                lt = (hi < hip) | (eq & (lo < lop))
                low = (pg & st) == 0
                asc = (pg & size) == 0
                eqm = asc == low
                take = (eqm & gt) | (~eqm & lt)
                hi = jnp.where(take, hip, hi)
                lo = jnp.where(take, lop, lo)
                idv = jnp.where(take, idp, idv)
                st //= 2
            size *= 2

        gsort = jnp.right_shift(hi, 24)
        sel_acc = jnp.zeros((128, 128), F32)
        pos_acc = jnp.full((128, 128), N, I32)
        sstart = jnp.int32(0)
        nstart = jnp.int32(0)
        for gi in range(G):
            c = jnp.sum(jnp.where(bkey == gi, jnp.int32(1), jnp.int32(0)))
            k = (c + 1) // 2
            m = gsort == gi
            rank = pg - sstart
            selm = m & (rank < k)
            sel_acc = jnp.where(selm, 1.0, sel_acc)
            pos_acc = jnp.where(selm, nstart + rank, pos_acc)
            sstart = sstart + c
            nstart = nstart + k
        sels_ref[...] = sel_acc
        poss_ref[...] = pos_acc
        ids_ref[...] = idv

    return pl.pallas_call(
        body,
        out_shape=[SDS((128, 128), F32),
                   SDS((128, 128), I32), SDS((128, 128), I32)],
    )(score_g, valid_g, pos_g, batch_g)


_PBR = 512


def _tc_pool(xp, score_col, sel_col, batch_col):
    nblk = NP // _PBR

    def body(xp_ref, sc_ref, sel_ref, b_ref, xn_ref, hv_ref, hm, hsm):
        i = pl.program_id(0)

        @pl.when(i == 0)
        def _():
            hm[...] = jnp.full((G, 128), -jnp.inf, F32)
            hsm[...] = jnp.zeros((G, 128), F32)

        xn = xp_ref[...] * sc_ref[...] * sel_ref[...]
        xn_ref[...] = xn
        selb = sel_ref[...] > 0.0
        onehot = jnp.where(
            (b_ref[...] == lax.broadcasted_iota(I32, (_PBR, G), 1)) & selb,
            1.0, 0.0)
        hsm[...] += lax.dot_general(onehot, xn, (((0,), (0,)), ((), ())),
                                    precision=HIGH, preferred_element_type=F32)
        for gi in range(G):
            rm = (b_ref[...] == gi) & selb
            vals = jnp.where(rm, xn, -jnp.inf)
            mx = jnp.max(vals, axis=0, keepdims=True)
            hm[gi:gi + 1, :] = jnp.maximum(hm[gi:gi + 1, :], mx)

        @pl.when(i == nblk - 1)
        def _():
            hv_ref[:, :128] = hm[...]
            hv_ref[:, 128:] = hsm[...]

    return pl.pallas_call(
        body,
        grid=(nblk,),
        in_specs=[
            pl.BlockSpec((_PBR, 128), lambda i: (i, 0)),
            pl.BlockSpec((_PBR, 1), lambda i: (i, 0)),
            pl.BlockSpec((_PBR, 1), lambda i: (i, 0)),
            pl.BlockSpec((_PBR, 1), lambda i: (i, 0)),
        ],
        out_specs=[
            pl.BlockSpec((_PBR, 128), lambda i: (i, 0)),
            pl.BlockSpec((G, 256), lambda i: (0, 0)),
        ],
        out_shape=[SDS((NP, 128), F32), SDS((G, 256), F32)],
        scratch_shapes=[pltpu.VMEM((G, 128), F32), pltpu.VMEM((G, 128), F32)],
    )(xp, score_col, sel_col, batch_col)


def _tc_mlp(h0, h1, h2, hls, W0a, W0b, bm0, Wm1, bm1, Wm2, bm2):
    def body(h0_ref, h1_ref, h2_ref, hls_ref, w0a_ref, w0b_ref, b0_ref,
             w1_ref, b1_ref, w2_ref, b2_ref, out_ref):
        h = h0_ref[...] + h1_ref[...] + h2_ref[...]
        z = jnp.dot(h, w0a_ref[...], preferred_element_type=F32)
        z = z + jnp.dot(hls_ref[...], w0b_ref[...],
                        preferred_element_type=F32)
        z = jnp.maximum(z + b0_ref[...], 0.0)
        z = jnp.maximum(jnp.dot(z, w1_ref[...],
                                preferred_element_type=F32) + b1_ref[...], 0.0)
        out_ref[...] = jnp.dot(z, w2_ref[...],
                               preferred_element_type=F32) + b2_ref[...]

    return pl.pallas_call(
        body,
        out_shape=SDS((G, 1), F32),
    )(h0, h1, h2, hls, W0a, W0b, bm0, Wm1, bm1, Wm2, bm2)


def kernel(x, edge_index, batch, hls_attr,
           Wg0, bg0, Wr0, br0, Wt0,
           Wg1, bg1, Wr1, br1, Wt1,
           Wg2, bg2, Wr2, br2, Wt2,
           Wm0, bm0, Wm1, bm1, Wm2, bm2):
    Wg = [Wg0, Wg1, Wg2]
    bg = [bg0, bg1, bg2]
    Wr = [Wr0, Wr1, Wr2]
    br = [br0, br1, br2]
    Wt = [Wt0, Wt1, Wt2]

    sc_edge_prep = _make_sc_edge_prep()
    sc_rowagg = _make_sc_rowagg()
    sc_scatter = _make_sc_scatter()

    batch_i = jnp.asarray(batch).astype(I32)
    src0 = jnp.concatenate(
        [edge_index[0].astype(I32), jnp.full((EP - E,), DUMMY, I32)]
    ).reshape(NW, ECH, 128)
    dst0 = jnp.concatenate(
        [edge_index[1].astype(I32), jnp.full((EP - E,), DUMMY, I32)]
    ).reshape(NW, ECH, 128)
    zeros_np = jnp.zeros((NP,), F32)
    zeros2d = jnp.zeros((128, 128), F32)
    sel01 = jnp.concatenate([jnp.ones((N,), F32), jnp.zeros((NP - N,), F32)])
    posn = jnp.concatenate(
        [jnp.arange(N, dtype=I32), jnp.full((NP - N,), N, I32)])
    batch_g = jnp.concatenate(
        [batch_i, jnp.full((SORTN - N,), G, I32)]).reshape(128, 128)
    batch_col = jnp.concatenate(
        [batch_i, jnp.full((NP - N,), G, I32)]).reshape(NP, 1)

    xcur = jnp.concatenate([x, jnp.zeros((NP - N, 128), F32)], axis=0)
    srcr, dstr = src0, dst0
    hs = []
    for l in range(3):
        srcr, dstr, deg = sc_edge_prep(sel01, srcr, dstr, zeros_np)
        y, dis = _tc_xw_y(xcur, Wg[l], deg.reshape(NC, NP, 1))
        acc = sc_rowagg(y, srcr, dstr, zeros2d)
        xp, u = _tc_x2(acc, y, dis, bg[l].reshape(1, 128), Wt[l])
        agg = sc_rowagg(xp, srcr, dstr, zeros2d)
        score_col = _tc_score(agg, Wr[l], u, br[l].reshape(1, 1))
        score_g = jnp.concatenate(
            [score_col.reshape(NP), jnp.zeros((SORTN - NP,), F32)]
        ).reshape(128, 128)
        valid_g = jnp.concatenate(
            [sel01, jnp.zeros((SORTN - NP,), F32)]).reshape(128, 128)
        pos_g = jnp.concatenate(
            [posn, jnp.full((SORTN - NP,), N, I32)]).reshape(128, 128)
        sel_s, pos_s, ids = _tc_sort(score_g, valid_g, pos_g, batch_g)
        sel01, posn = sc_scatter(
            ids.reshape(NW, 4, 128), sel_s.reshape(NW, 4, 128),
            pos_s.reshape(NW, 4, 128))
        xcur, hv = _tc_pool(xp, score_col, sel01.reshape(NP, 1), batch_col)
        hs.append(hv)

    return _tc_mlp(hs[0], hs[1], hs[2], hls_attr,
                   Wm0[:256], Wm0[256:], bm0.reshape(1, 64),
                   Wm1, bm1.reshape(1, 64), Wm2, bm2.reshape(1, 1))


# speedup vs baseline: 1.0653x; 1.0628x over previous
import functools

import jax
import jax.numpy as jnp
from jax import lax
from jax.experimental import pallas as pl
from jax.experimental.pallas import tpu as pltpu
from jax.experimental.pallas import tpu_sc as plsc

N = 10000
NP = 10240
E = 320000
G = 64
NC, NS = 2, 16
NW = NC * NS
EPW = 10240
ECH = EPW // 128
EP = NW * EPW
DUMMY = N
SORTN = 16384
SLICE = NP // NS
F32, I32 = jnp.float32, jnp.int32
HIGH = lax.Precision.HIGHEST
SDS = jax.ShapeDtypeStruct


def _mesh():
    return plsc.VectorSubcoreMesh(
        core_axis_name="c", subcore_axis_name="s", num_cores=NC, num_subcores=NS)


def _make_sc_edge_prep():
    @functools.partial(
        pl.kernel,
        out_type=(SDS((NW, ECH, 128), I32), SDS((NW, ECH, 128), I32),
                  SDS((NC, NP), F32)),
        mesh=_mesh(),
        scratch_types=[
            pltpu.VMEM((ECH, 128), I32),
            pltpu.VMEM((ECH, 128), I32),
            pltpu.VMEM((EPW,), F32),
            pltpu.VMEM((EPW,), F32),
            pltpu.VMEM((EPW,), F32),
            pltpu.VMEM((SLICE,), F32),
            pltpu.VMEM_SHARED((NP,), F32),
            pltpu.SemaphoreType.DMA,
            pltpu.SemaphoreType.DMA,
        ],
    )
    def sc_edge_prep(sel_hbm, src_hbm, dst_hbm, zeros_hbm,
                     srco_hbm, dsto_hbm, deg_hbm,
                     src_v, dst_v, kbuf, ssbuf, sdbuf, zbuf, deg_s,
                     sem_a, sem_b):
        cid = lax.axis_index("c")
        sid = lax.axis_index("s")
        wid = sid * NC + cid
        pltpu.sync_copy(src_hbm.at[wid], src_v)
        pltpu.sync_copy(dst_hbm.at[wid], dst_v)
        pltpu.sync_copy(zeros_hbm.at[pl.ds(sid * SLICE, SLICE)], zbuf)
        pltpu.sync_copy(zbuf, deg_s.at[pl.ds(sid * SLICE, SLICE)])
        plsc.subcore_barrier()

        def fire(j, carry):
            pltpu.async_copy(sel_hbm.at[src_v.at[j]],
                             ssbuf.at[pl.ds(j * 128, 128)], sem_a)
            pltpu.async_copy(sel_hbm.at[dst_v.at[j]],
                             sdbuf.at[pl.ds(j * 128, 128)], sem_b)
            return carry

        lax.fori_loop(0, ECH, fire, 0)
        pltpu.make_async_copy(zeros_hbm, ssbuf, sem_a).wait()
        pltpu.make_async_copy(zeros_hbm, sdbuf, sem_b).wait()

        def chunk(j, carry):
            for b in range(8):
                sl = pl.ds(b * 16, 16)
                fl = pl.ds(j * 128 + b * 16, 16)
                s = src_v[j, sl]
                d = dst_v[j, sl]
                keep = ssbuf[fl] * sdbuf[fl]
                km = keep > 0.0
                src_v[j, sl] = jnp.where(km, s, DUMMY)
                dst_v[j, sl] = jnp.where(km, d, DUMMY)
                kbuf[fl] = keep
            return carry

        lax.fori_loop(0, ECH, chunk, 0)

        def fire2(j, carry):
            pltpu.async_copy(kbuf.at[pl.ds(j * 128, 128)],
                             deg_s.at[dst_v.at[j]], sem_a, add=True)
            return carry

        lax.fori_loop(0, ECH, fire2, 0)
        pltpu.make_async_copy(zeros_hbm, kbuf, sem_a).wait()
        plsc.subcore_barrier()
        pltpu.sync_copy(src_v, srco_hbm.at[wid])
        pltpu.sync_copy(dst_v, dsto_hbm.at[wid])
        pltpu.sync_copy(deg_s.at[pl.ds(sid * SLICE, SLICE)], zbuf)
        pltpu.sync_copy(zbuf, deg_hbm.at[cid, pl.ds(sid * SLICE, SLICE)])

    return sc_edge_prep


def _make_sc_rowagg():
    @functools.partial(
        pl.kernel,
        out_type=SDS((NC, NP, 128), F32),
        mesh=_mesh(),
        scratch_types=[
            pltpu.VMEM((ECH, 128), I32),
            pltpu.VMEM((1, 128), I32),
            pltpu.VMEM((1, 128), I32),
            pltpu.VMEM((128, 128), F32),
            pltpu.VMEM((128, 128), F32),
            pltpu.VMEM_SHARED((NP, 128), F32),
            pltpu.SemaphoreType.DMA,
            pltpu.SemaphoreType.DMA,
        ],
    )
    def sc_rowagg(y_hbm, src_hbm, dst_hbm, zeros2d_hbm, acc_hbm,
                  src_v, dst_c0, dst_c1, rows0, rows1, acc_s, sem_g0, sem_g1):
        cid = lax.axis_index("c")
        sid = lax.axis_index("s")
        wid = sid * NC + cid
        pltpu.sync_copy(src_hbm.at[wid], src_v)
        pltpu.sync_copy(zeros2d_hbm, rows0)
        for q in range(SLICE // 128):
            pltpu.sync_copy(rows0, acc_s.at[pl.ds(sid * SLICE + q * 128, 128)])
        plsc.subcore_barrier()
        pltpu.async_copy(y_hbm.at[src_v.at[0]], rows0, sem_g0)
        pltpu.sync_copy(dst_hbm.at[wid, pl.ds(0, 1)], dst_c0)

        def pair(g, carry):
            j0 = 2 * g
            j1 = 2 * g + 1
            pltpu.async_copy(y_hbm.at[src_v.at[j1]], rows1, sem_g1)
            pltpu.sync_copy(dst_hbm.at[wid, pl.ds(j1, 1)], dst_c1)
            pltpu.make_async_copy(y_hbm.at[src_v.at[j0]], rows0, sem_g0).wait()
            pltpu.sync_copy(rows0, acc_s.at[dst_c0.at[0]], add=True)

            @pl.when(j0 + 2 < ECH)
            def _():
                pltpu.async_copy(y_hbm.at[src_v.at[j0 + 2]], rows0, sem_g0)
                pltpu.sync_copy(dst_hbm.at[wid, pl.ds(j0 + 2, 1)], dst_c0)

            pltpu.make_async_copy(y_hbm.at[src_v.at[j1]], rows1, sem_g1).wait()
            pltpu.sync_copy(rows1, acc_s.at[dst_c1.at[0]], add=True)
            return carry

        lax.fori_loop(0, ECH // 2, pair, 0)
        plsc.subcore_barrier()
        for q in range(SLICE // 128):
            r0 = sid * SLICE + q * 128
            pltpu.sync_copy(acc_s.at[pl.ds(r0, 128)], rows0)
            pltpu.sync_copy(rows0, acc_hbm.at[cid, pl.ds(r0, 128)])

    return sc_rowagg


def _make_sc_scatter():
    @functools.partial(
        pl.kernel,
        out_type=(SDS((NP,), F32), SDS((NP,), I32)),
        mesh=_mesh(),
        scratch_types=[
            pltpu.VMEM((4, 128), I32),
            pltpu.VMEM((4, 128), F32),
            pltpu.VMEM((4, 128), I32),
            pltpu.SemaphoreType.DMA,
            pltpu.SemaphoreType.DMA,
        ],
    )
    def sc_scatter(ids_hbm, sel_hbm_in, pos_hbm_in, sel_hbm, pos_hbm,
                   id_v, sv, pv, sem_a, sem_b):
        cid = lax.axis_index("c")
        sid = lax.axis_index("s")
        wid = sid * NC + cid
        pltpu.sync_copy(ids_hbm.at[wid], id_v)
        pltpu.sync_copy(sel_hbm_in.at[wid], sv)
        pltpu.sync_copy(pos_hbm_in.at[wid], pv)
        for q in range(4):
            pltpu.async_copy(sv.at[q], sel_hbm.at[id_v.at[q]], sem_a)
            pltpu.async_copy(pv.at[q], pos_hbm.at[id_v.at[q]], sem_b)
        for q in range(4):
            pltpu.make_async_copy(sv.at[q], sel_hbm.at[id_v.at[q]], sem_a).wait()
            pltpu.make_async_copy(pv.at[q], pos_hbm.at[id_v.at[q]], sem_b).wait()

    return sc_scatter


_BR = 1024


def _tc_xw_y(x, W, deg2):
    def body(x_ref, w_ref, deg_ref, y_ref, dis_ref):
        i = pl.program_id(0)
        deg = deg_ref[0] + deg_ref[1] + 1.0
        dis = 1.0 / jnp.sqrt(deg)
        xw = jnp.dot(x_ref[...], w_ref[...], preferred_element_type=F32)
        rowid = i * _BR + lax.broadcasted_iota(I32, (_BR, 1), 0)
        y_ref[...] = jnp.where(rowid < N, xw * dis, 0.0)
        dis_ref[...] = dis

    return pl.pallas_call(
        body,
        grid=(NP // _BR,),
        in_specs=[
            pl.BlockSpec((_BR, 128), lambda i: (i, 0)),
            pl.BlockSpec((128, 128), lambda i: (0, 0)),
            pl.BlockSpec((2, _BR, 1), lambda i: (0, i, 0)),
        ],
        out_specs=[
            pl.BlockSpec((_BR, 128), lambda i: (i, 0)),
            pl.BlockSpec((_BR, 1), lambda i: (i, 0)),
        ],
        out_shape=[SDS((NP, 128), F32), SDS((NP, 1), F32)],
    )(x, W, deg2)


def _tc_x2(acc, y, dis, bg, Wt):
    def body(acc_ref, y_ref, dis_ref, bg_ref, wt_ref, xp_ref, u_ref):
        i = pl.program_id(0)
        s = acc_ref[0] + acc_ref[1] + y_ref[...]
        xp = jnp.maximum(dis_ref[...] * s + bg_ref[...], 0.0)
        rowid = i * _BR + lax.broadcasted_iota(I32, (_BR, 1), 0)
        xp = jnp.where(rowid < N, xp, 0.0)
        xp_ref[...] = xp
        u_ref[...] = jnp.dot(xp, wt_ref[...], preferred_element_type=F32)

    return pl.pallas_call(
        body,
        grid=(NP // _BR,),
        in_specs=[
            pl.BlockSpec((2, _BR, 128), lambda i: (0, i, 0)),
            pl.BlockSpec((_BR, 128), lambda i: (i, 0)),
            pl.BlockSpec((_BR, 1), lambda i: (i, 0)),
            pl.BlockSpec((1, 128), lambda i: (0, 0)),
            pl.BlockSpec((128, 1), lambda i: (0, 0)),
        ],
        out_specs=[
            pl.BlockSpec((_BR, 128), lambda i: (i, 0)),
            pl.BlockSpec((_BR, 1), lambda i: (i, 0)),
        ],
        out_shape=[SDS((NP, 128), F32), SDS((NP, 1), F32)],
    )(acc, y, dis, bg, Wt)


def _tc_score(agg, Wr, u, br):
    def body(agg_ref, wr_ref, u_ref, br_ref, sc_ref):
        a = agg_ref[0] + agg_ref[1]
        d = jnp.dot(a, wr_ref[...], preferred_element_type=F32)
        sc_ref[...] = jnp.tanh(d + br_ref[0, 0] + u_ref[...])

    return pl.pallas_call(
        body,
        grid=(NP // _BR,),
        in_specs=[
            pl.BlockSpec((2, _BR, 128), lambda i: (0, i, 0)),
            pl.BlockSpec((128, 1), lambda i: (0, 0)),
            pl.BlockSpec((_BR, 1), lambda i: (i, 0)),
            pl.BlockSpec((1, 1), lambda i: (0, 0)),
        ],
        out_specs=[pl.BlockSpec((_BR, 1), lambda i: (i, 0))],
        out_shape=[SDS((NP, 1), F32)],
    )(agg, Wr, u, br)[0]


def _tc_sort(score_g, valid_g, pos_g, batch_g):
    def body(score_ref, val_ref, pos_ref, bat_ref, sels_ref, poss_ref):
        score = score_ref[...]
        validb = val_ref[...] > 0.0
        bkey = jnp.where(validb, bat_ref[...], G)
        skey = jnp.where(validb, -score, jnp.inf) + 0.0
        bits = lax.bitcast_convert_type(skey, I32)
        o = jnp.where(bits >= 0, bits ^ jnp.int32(-2147483648), ~bits)
        hi = (bkey << 24) | (jnp.right_shift(o, 8) & jnp.int32(0xFFFFFF))
        lo = ((o & jnp.int32(0xFF)) << 14) | pos_ref[...]
        rows = lax.broadcasted_iota(I32, (128, 128), 0)
        cols = lax.broadcasted_iota(I32, (128, 128), 1)
        pg = rows * 128 + cols
        idv = jnp.minimum(pg, NP - 1)

        def xorperm(x, st):
            if st >= 128:
                m = st // 128
                up = jnp.concatenate([x[m:, :], x[:m, :]], axis=0)
                dn = jnp.concatenate([x[128 - m:, :], x[:128 - m, :]], axis=0)
                bitc = (rows & m) == 0
            else:
                m = st
                up = jnp.concatenate([x[:, m:], x[:, :m]], axis=1)
                dn = jnp.concatenate([x[:, 128 - m:], x[:, :128 - m]], axis=1)
                bitc = (cols & m) == 0
            return jnp.where(bitc, up, dn)

        size = 2
        while size <= SORTN:
            st = size // 2
            while st >= 1:
                hip = xorperm(hi, st)
                lop = xorperm(lo, st)
                idp = xorperm(idv, st)
                eq = hi == hip
                gt = (hi > hip) | (eq & (lo > lop))
                lt = (hi < hip) | (eq & (lo < lop))
                low = (pg & st) == 0
                asc = (pg & size) == 0
                eqm = asc == low
                take = (eqm & gt) | (~eqm & lt)
                hi = jnp.where(take, hip, hi)
                lo = jnp.where(take, lop, lo)
                idv = jnp.where(take, idp, idv)
                st //= 2
            size *= 2

        gsort = jnp.right_shift(hi, 24)
        sel_acc = jnp.zeros((128, 128), F32)
        pos_acc = jnp.full((128, 128), N, I32)
        sstart = jnp.int32(0)
        nstart = jnp.int32(0)
        for gi in range(G):
            c = jnp.sum(jnp.where(bkey == gi, jnp.int32(1), jnp.int32(0)))
            k = (c + 1) // 2
            m = gsort == gi
            rank = pg - sstart
            selm = m & (rank < k)
            sel_acc = jnp.where(selm, 1.0, sel_acc)
            pos_acc = jnp.where(selm, nstart + rank, pos_acc)
            sstart = sstart + c
            nstart = nstart + k

        selb32 = lax.bitcast_convert_type(sel_acc, I32)
        size = 2
        while size <= SORTN:
            st = size // 2
            while st >= 1:
                kp = xorperm(idv, st)
                sp = xorperm(selb32, st)
                pp = xorperm(pos_acc, st)
                gt = idv > kp
                lt = idv < kp
                low = (pg & st) == 0
                asc = (pg & size) == 0
                eqm = asc == low
                take = (eqm & gt) | (~eqm & lt)
                idv = jnp.where(take, kp, idv)
                selb32 = jnp.where(take, sp, selb32)
                pos_acc = jnp.where(take, pp, pos_acc)
                st //= 2
            size *= 2
        sels_ref[...] = lax.bitcast_convert_type(selb32, F32)
        poss_ref[...] = pos_acc

    return pl.pallas_call(
        body,
        out_shape=[SDS((128, 128), F32), SDS((128, 128), I32)],
    )(score_g, valid_g, pos_g, batch_g)


_PBR = 512


def _tc_pool(xp, score_col, sel_col, batch_col):
    nblk = NP // _PBR

    def body(xp_ref, sc_ref, sel_ref, b_ref, xn_ref, hv_ref, hm, hsm):
        i = pl.program_id(0)

        @pl.when(i == 0)
        def _():
            hm[...] = jnp.full((G, 128), -jnp.inf, F32)
            hsm[...] = jnp.zeros((G, 128), F32)

        xn = xp_ref[...] * sc_ref[...] * sel_ref[...]
        xn_ref[...] = xn
        selb = sel_ref[...] > 0.0
        onehot = jnp.where(
            (b_ref[...] == lax.broadcasted_iota(I32, (_PBR, G), 1)) & selb,
            1.0, 0.0)
        hsm[...] += lax.dot_general(onehot, xn, (((0,), (0,)), ((), ())),
                                    precision=HIGH, preferred_element_type=F32)
        for gi in range(G):
            rm = (b_ref[...] == gi) & selb
            vals = jnp.where(rm, xn, -jnp.inf)
            mx = jnp.max(vals, axis=0, keepdims=True)
            hm[gi:gi + 1, :] = jnp.maximum(hm[gi:gi + 1, :], mx)

        @pl.when(i == nblk - 1)
        def _():
            hv_ref[:, :128] = hm[...]
            hv_ref[:, 128:] = hsm[...]

    return pl.pallas_call(
        body,
        grid=(nblk,),
        in_specs=[
            pl.BlockSpec((_PBR, 128), lambda i: (i, 0)),
            pl.BlockSpec((_PBR, 1), lambda i: (i, 0)),
            pl.BlockSpec((_PBR, 1), lambda i: (i, 0)),
            pl.BlockSpec((_PBR, 1), lambda i: (i, 0)),
        ],
        out_specs=[
            pl.BlockSpec((_PBR, 128), lambda i: (i, 0)),
            pl.BlockSpec((G, 256), lambda i: (0, 0)),
        ],
        out_shape=[SDS((NP, 128), F32), SDS((G, 256), F32)],
        scratch_shapes=[pltpu.VMEM((G, 128), F32), pltpu.VMEM((G, 128), F32)],
    )(xp, score_col, sel_col, batch_col)


def _tc_mlp(h0, h1, h2, hls, W0a, W0b, bm0, Wm1, bm1, Wm2, bm2):
    def body(h0_ref, h1_ref, h2_ref, hls_ref, w0a_ref, w0b_ref, b0_ref,
             w1_ref, b1_ref, w2_ref, b2_ref, out_ref):
        h = h0_ref[...] + h1_ref[...] + h2_ref[...]
        z = jnp.dot(h, w0a_ref[...], preferred_element_type=F32)
        z = z + jnp.dot(hls_ref[...], w0b_ref[...],
                        preferred_element_type=F32)
        z = jnp.maximum(z + b0_ref[...], 0.0)
        z = jnp.maximum(jnp.dot(z, w1_ref[...],
                                preferred_element_type=F32) + b1_ref[...], 0.0)
        out_ref[...] = jnp.dot(z, w2_ref[...],
                               preferred_element_type=F32) + b2_ref[...]

    return pl.pallas_call(
        body,
        out_shape=SDS((G, 1), F32),
    )(h0, h1, h2, hls, W0a, W0b, bm0, Wm1, bm1, Wm2, bm2)


def kernel(x, edge_index, batch, hls_attr,
           Wg0, bg0, Wr0, br0, Wt0,
           Wg1, bg1, Wr1, br1, Wt1,
           Wg2, bg2, Wr2, br2, Wt2,
           Wm0, bm0, Wm1, bm1, Wm2, bm2):
    Wg = [Wg0, Wg1, Wg2]
    bg = [bg0, bg1, bg2]
    Wr = [Wr0, Wr1, Wr2]
    br = [br0, br1, br2]
    Wt = [Wt0, Wt1, Wt2]

    sc_edge_prep = _make_sc_edge_prep()
    sc_rowagg = _make_sc_rowagg()

    batch_i = jnp.asarray(batch).astype(I32)
    src0 = jnp.concatenate(
        [edge_index[0].astype(I32), jnp.full((EP - E,), DUMMY, I32)]
    ).reshape(NW, ECH, 128)
    dst0 = jnp.concatenate(
        [edge_index[1].astype(I32), jnp.full((EP - E,), DUMMY, I32)]
    ).reshape(NW, ECH, 128)
    zeros_np = jnp.zeros((NP,), F32)
    zeros2d = jnp.zeros((128, 128), F32)
    sel01 = jnp.concatenate([jnp.ones((N,), F32), jnp.zeros((NP - N,), F32)])
    posn = jnp.concatenate(
        [jnp.arange(N, dtype=I32), jnp.full((NP - N,), N, I32)])
    batch_g = jnp.concatenate(
        [batch_i, jnp.full((SORTN - N,), G, I32)]).reshape(128, 128)
    batch_col = jnp.concatenate(
        [batch_i, jnp.full((NP - N,), G, I32)]).reshape(NP, 1)

    xcur = jnp.concatenate([x, jnp.zeros((NP - N, 128), F32)], axis=0)
    srcr, dstr = src0, dst0
    hs = []
    for l in range(3):
        srcr, dstr, deg = sc_edge_prep(sel01, srcr, dstr, zeros_np)
        y, dis = _tc_xw_y(xcur, Wg[l], deg.reshape(NC, NP, 1))
        acc = sc_rowagg(y, srcr, dstr, zeros2d)
        xp, u = _tc_x2(acc, y, dis, bg[l].reshape(1, 128), Wt[l])
        agg = sc_rowagg(xp, srcr, dstr, zeros2d)
        score_col = _tc_score(agg, Wr[l], u, br[l].reshape(1, 1))
        score_g = jnp.concatenate(
            [score_col.reshape(NP), jnp.zeros((SORTN - NP,), F32)]
        ).reshape(128, 128)
        valid_g = jnp.concatenate(
            [sel01, jnp.zeros((SORTN - NP,), F32)]).reshape(128, 128)
        pos_g = jnp.concatenate(
            [posn, jnp.full((SORTN - NP,), N, I32)]).reshape(128, 128)
        sel_n, pos_n = _tc_sort(score_g, valid_g, pos_g, batch_g)
        sel01 = sel_n.reshape(SORTN)[:NP]
        posn = pos_n.reshape(SORTN)[:NP]
        xcur, hv = _tc_pool(xp, score_col, sel01.reshape(NP, 1), batch_col)
        hs.append(hv)

    return _tc_mlp(hs[0], hs[1], hs[2], hls_attr,
                   Wm0[:256], Wm0[256:], bm0.reshape(1, 64),
                   Wm1, bm1.reshape(1, 64), Wm2, bm2.reshape(1, 1))
